# Initial kernel scaffold; baseline (speedup 1.0000x reference)
import jax
import jax.numpy as jnp
from jax.experimental import pallas as pl


def kernel(x, weight_train, weight_freeze):
    raise NotImplementedError("write your pallas kernel here")


# speedup vs baseline: 1.6066x; 1.6066x over previous
import functools

import jax
import jax.numpy as jnp
from jax import lax
from jax.experimental import pallas as pl
from jax.experimental.pallas import tpu as pltpu
from jax.experimental.pallas import tpu_sc as plsc

D = 32
L = 16
GATHER = 128


@functools.lru_cache(maxsize=None)
def _make_kernel(B, C):
    NC, NS = 2, 16
    NW = NC * NS
    N = B // NW
    G = C // GATHER
    NCH = N // C
    assert B % (NW * C) == 0 and C % GATHER == 0

    mesh = plsc.VectorSubcoreMesh(core_axis_name="c", subcore_axis_name="s")

    @functools.partial(
        pl.kernel,
        mesh=mesh,
        out_type=jax.ShapeDtypeStruct((B, D), jnp.float32),
        compiler_params=pltpu.CompilerParams(
            use_tc_tiling_on_sc=False, needs_layout_passes=False
        ),
        scratch_types=[
            pltpu.VMEM((G, GATHER), jnp.int32),
            pltpu.VMEM((G, GATHER), jnp.int32),
            pltpu.VMEM((C, D), jnp.float32),
            pltpu.VMEM((D,), jnp.float32),
            pltpu.SemaphoreType.DMA,
        ],
    )
    def emb(idx_hbm, train_hbm, freeze_hbm, out_hbm, idx_v, adj_v, rows_v, fz_v, gsem):
        wid = lax.axis_index("s") * NC + lax.axis_index("c")
        pltpu.sync_copy(freeze_hbm.at[0], fz_v)
        base = wid * (N // GATHER)

        for ch in range(NCH):
            rbase = base + ch * G
            pltpu.sync_copy(idx_hbm.at[pl.ds(rbase, G)], idx_v)

            def adjust_and_fire(j, za):
                for t in range(GATHER // L):
                    v = idx_v[j, pl.ds(t * L, L)]
                    za = jnp.logical_or(za, v == 0)
                    adj_v[j, pl.ds(t * L, L)] = jnp.maximum(v - 1, 0)
                pltpu.async_copy(
                    train_hbm.at[adj_v.at[j]],
                    rows_v.at[pl.ds(j * GATHER, GATHER)],
                    gsem,
                )
                return za

            za = lax.fori_loop(0, G, adjust_and_fire, jnp.zeros((L,), jnp.bool_))

            def drain(j, carry):
                pltpu.make_async_copy(
                    train_hbm.at[pl.ds(0, GATHER)],
                    rows_v.at[pl.ds(j * GATHER, GATHER)],
                    gsem,
                ).wait()
                return carry

            lax.fori_loop(0, G, drain, jnp.int32(0))

            @pl.when(jnp.any(za))
            def _():
                fz = [fz_v[pl.ds(k * L, L)] for k in range(D // L)]

                def fix(g, carry):
                    v = idx_v[g // (GATHER // L), pl.ds((g % (GATHER // L)) * L, L)]
                    m = v == 0

                    @pl.when(jnp.any(m))
                    def _():
                        rowi = g * L + lax.iota(jnp.int32, L)
                        for c in range(D):
                            colv = jnp.full((L,), c, jnp.int32)
                            val = jnp.full((L,), fz[c // L][c % L], jnp.float32)
                            plsc.store_scatter(rows_v, [rowi, colv], val, mask=m)

                    return carry

                lax.fori_loop(0, C // L, fix, jnp.int32(0))

            pltpu.sync_copy(rows_v, out_hbm.at[pl.ds(rbase * GATHER, C)])

    return emb


def kernel(x, weight_train, weight_freeze):
    B = x.size
    xf = x.reshape(B // GATHER, GATHER).astype(jnp.int32)
    out = _make_kernel(B, 2048)(xf, weight_train, weight_freeze)
    return out.reshape(*x.shape, D)


# speedup vs baseline: 1.6196x; 1.0081x over previous
import functools

import jax
import jax.numpy as jnp
from jax import lax
from jax.experimental import pallas as pl
from jax.experimental.pallas import tpu as pltpu
from jax.experimental.pallas import tpu_sc as plsc

D = 32
L = 16
GATHER = 128
NB = 2


@functools.lru_cache(maxsize=None)
def _make_kernel(B, C):
    NC, NS = 2, 16
    NW = NC * NS
    N = B // NW
    CG = C // GATHER
    NCH = N // C
    GA = N // GATHER
    assert B % (NW * C) == 0 and C % GATHER == 0

    mesh = plsc.VectorSubcoreMesh(core_axis_name="c", subcore_axis_name="s")

    @functools.partial(
        pl.kernel,
        mesh=mesh,
        out_type=jax.ShapeDtypeStruct((B, D), jnp.float32),
        compiler_params=pltpu.CompilerParams(
            use_tc_tiling_on_sc=False, needs_layout_passes=False
        ),
        scratch_types=[
            pltpu.VMEM((GA, GATHER), jnp.int32),
            pltpu.VMEM((GA, GATHER), jnp.int32),
            pltpu.VMEM((NB, C, D), jnp.float32),
            pltpu.VMEM((D,), jnp.float32),
            pltpu.SemaphoreType.DMA,
            pltpu.SemaphoreType.DMA,
            pltpu.SemaphoreType.DMA,
            pltpu.SemaphoreType.DMA,
        ],
    )
    def emb(idx_hbm, train_hbm, freeze_hbm, out_hbm,
            idx_v, adj_v, rows_v, fz_v, gsem0, gsem1, wsem0, wsem1):
        gsem = [gsem0, gsem1]
        wsem = [wsem0, wsem1]
        wid = lax.axis_index("s") * NC + lax.axis_index("c")
        base = wid * GA
        pltpu.sync_copy(freeze_hbm.at[0], fz_v)
        pltpu.sync_copy(idx_hbm.at[pl.ds(base, GA)], idx_v)

        def adjust(c):
            def body(j, za):
                for t in range(GATHER // L):
                    v = idx_v[j, pl.ds(t * L, L)]
                    za = jnp.logical_or(za, v == 0)
                    adj_v[j, pl.ds(t * L, L)] = jnp.maximum(v - 1, 0)
                return za
            return lax.fori_loop(c * CG, (c + 1) * CG, body,
                                 jnp.zeros((L,), jnp.bool_))

        def fire(c, b):
            for g in range(CG):
                j = c * CG + g
                pltpu.async_copy(
                    train_hbm.at[adj_v.at[j]],
                    rows_v.at[b].at[pl.ds(g * GATHER, GATHER)],
                    gsem[b],
                )

        def drain_gathers(b):
            for g in range(CG):
                pltpu.make_async_copy(
                    train_hbm.at[pl.ds(0, GATHER)],
                    rows_v.at[b].at[pl.ds(g * GATHER, GATHER)],
                    gsem[b],
                ).wait()

        def fix_zeros(c, b, za):
            @pl.when(jnp.any(za))
            def _():
                fz = [fz_v[pl.ds(k * L, L)] for k in range(D // L)]

                def body(g, carry):
                    v = idx_v[c * CG + g // (GATHER // L),
                              pl.ds((g % (GATHER // L)) * L, L)]
                    m = v == 0

                    @pl.when(jnp.any(m))
                    def _():
                        rowi = g * L + lax.iota(jnp.int32, L)
                        for col in range(D):
                            colv = jnp.full((L,), col, jnp.int32)
                            val = jnp.full((L,), fz[col // L][col % L], jnp.float32)
                            plsc.store_scatter(rows_v.at[b], [rowi, colv], val,
                                               mask=m)

                    return carry

                lax.fori_loop(0, C // L, body, jnp.int32(0))

        def writeback(c, b):
            pltpu.async_copy(
                rows_v.at[b],
                out_hbm.at[pl.ds((base + c * CG) * GATHER, C)],
                wsem[b],
            )

        def drain_writeback(c, b):
            pltpu.make_async_copy(
                rows_v.at[b],
                out_hbm.at[pl.ds((base + c * CG) * GATHER, C)],
                wsem[b],
            ).wait()

        zas = [None] * NCH
        for c in range(NCH):
            b = c % NB
            if c >= NB:
                drain_writeback(c - NB, b)
            zas[c] = adjust(c)
            fire(c, b)
            if c >= 1:
                pb = (c - 1) % NB
                drain_gathers(pb)
                fix_zeros(c - 1, pb, zas[c - 1])
                writeback(c - 1, pb)

        lb = (NCH - 1) % NB
        drain_gathers(lb)
        fix_zeros(NCH - 1, lb, zas[NCH - 1])
        writeback(NCH - 1, lb)
        for c in range(max(0, NCH - NB), NCH):
            drain_writeback(c, c % NB)

    return emb


def kernel(x, weight_train, weight_freeze):
    B = x.size
    xf = x.reshape(B // GATHER, GATHER).astype(jnp.int32)
    out = _make_kernel(B, 1024)(xf, weight_train, weight_freeze)
    return out.reshape(*x.shape, D)
